# Initial kernel scaffold; baseline (speedup 1.0000x reference)
import jax
import jax.numpy as jnp
from jax.experimental import pallas as pl


def kernel(input_features, table):
    raise NotImplementedError("write your pallas kernel here")


# speedup vs baseline: 5.4570x; 5.4570x over previous
import functools

import jax
import jax.numpy as jnp
from jax import lax
from jax.experimental import pallas as pl
from jax.experimental.pallas import tpu as pltpu
from jax.experimental.pallas import tpu_sc as plsc

N_FIELDS = 26
VALS_PER_FIELD = 32
OUTPUT_DIM = 64
BATCH = 16384
BF = BATCH * N_FIELDS

_info = plsc.get_sparse_core_info()
NC, NS, L = _info.num_cores, _info.num_subcores, _info.num_lanes
NW = NC * NS
PER_W = BF // NW
CHUNK = 128
N_CHUNKS = PER_W // CHUNK


@functools.partial(
    pl.kernel,
    mesh=plsc.VectorSubcoreMesh(core_axis_name="c", subcore_axis_name="s"),
    out_type=jax.ShapeDtypeStruct((BF, OUTPUT_DIM), jnp.float32),
    scratch_types=[
        pltpu.VMEM((CHUNK,), jnp.int32),
        pltpu.VMEM((CHUNK,), jnp.int32),
        pltpu.VMEM((CHUNK, OUTPUT_DIM), jnp.float32),
        pltpu.SemaphoreType.DMA,
    ],
    compiler_params=pltpu.CompilerParams(use_tc_tiling_on_sc=False),
)
def _sc_gather(in_hbm, table_hbm, out_hbm, vin, idxbuf, rows, sem):
    wid = lax.axis_index("s") * NC + lax.axis_index("c")
    base = wid * PER_W
    lane = lax.iota(jnp.int32, L)

    def body(c, _):
        start = base + c * CHUNK
        pltpu.sync_copy(in_hbm.at[pl.ds(start, CHUNK)], vin)
        for i in range(CHUNK // L):
            v = vin[pl.ds(i * L, L)]
            pos = start + i * L + lane
            field = lax.rem(pos, N_FIELDS)
            local = jnp.where((v >= 0) & (v < VALS_PER_FIELD), v, 0)
            idxbuf[pl.ds(i * L, L)] = local + field * VALS_PER_FIELD
        pltpu.async_copy(table_hbm.at[idxbuf], rows, sem).wait()
        pltpu.sync_copy(rows, out_hbm.at[pl.ds(start, CHUNK)])
        return _

    lax.fori_loop(0, N_CHUNKS, body, None)


def kernel(input_features, table):
    flat_in = input_features.reshape(BF)
    out = _sc_gather(flat_in, table)
    return out.reshape(BATCH, N_FIELDS * OUTPUT_DIM)


# speedup vs baseline: 6.0392x; 1.1067x over previous
import functools

import jax
import jax.numpy as jnp
from jax import lax
from jax.experimental import pallas as pl
from jax.experimental.pallas import tpu as pltpu
from jax.experimental.pallas import tpu_sc as plsc

N_FIELDS = 26
VALS_PER_FIELD = 32
OUTPUT_DIM = 64
BATCH = 16384
BF = BATCH * N_FIELDS

_info = plsc.get_sparse_core_info()
NC, NS, L = _info.num_cores, _info.num_subcores, _info.num_lanes
NW = NC * NS
PER_W = BF // NW
CHUNK = 128
N_CHUNKS = PER_W // CHUNK
NBUF = 4
SKEW = 2


@functools.partial(
    pl.kernel,
    mesh=plsc.VectorSubcoreMesh(core_axis_name="c", subcore_axis_name="s"),
    out_type=jax.ShapeDtypeStruct((BF, OUTPUT_DIM), jnp.float32),
    scratch_types=[
        pltpu.VMEM((N_CHUNKS, CHUNK), jnp.int32),
        pltpu.VMEM((NBUF, CHUNK, OUTPUT_DIM), jnp.float32),
        pltpu.SemaphoreType.DMA,
    ]
    + [pltpu.SemaphoreType.DMA] * NBUF
    + [pltpu.SemaphoreType.DMA] * NBUF,
    compiler_params=pltpu.CompilerParams(use_tc_tiling_on_sc=False),
)
def _sc_gather(in_hbm, table_hbm, out_hbm, vidx, rows, insem, *sems):
    gsems = sems[:NBUF]
    ssems = sems[NBUF:]
    wid = lax.axis_index("s") * NC + lax.axis_index("c")
    base = wid * PER_W
    lane = lax.iota(jnp.int32, L)

    pltpu.async_copy(in_hbm.at[wid], vidx, insem).wait()

    def cbody(c, _):
        for i in range(CHUNK // L):
            v = vidx[c, pl.ds(i * L, L)]
            pos = base + c * CHUNK + i * L + lane
            field = lax.rem(pos, N_FIELDS)
            local = jnp.where((v >= 0) & (v < VALS_PER_FIELD), v, 0)
            vidx[c, pl.ds(i * L, L)] = local + field * VALS_PER_FIELD
        return _

    lax.fori_loop(0, N_CHUNKS, cbody, None)

    ghandles = [None] * N_CHUNKS
    shandles = [None] * N_CHUNKS

    def scatter(d):
        s = d % NBUF
        ghandles[d].wait()
        shandles[d] = pltpu.async_copy(
            rows.at[s], out_hbm.at[pl.ds(base + d * CHUNK, CHUNK)], ssems[s]
        )

    for c in range(N_CHUNKS):
        s = c % NBUF
        if c >= NBUF:
            shandles[c - NBUF].wait()
        ghandles[c] = pltpu.async_copy(table_hbm.at[vidx.at[c]], rows.at[s], gsems[s])
        if c - SKEW >= 0:
            scatter(c - SKEW)
    for d in range(N_CHUNKS - SKEW, N_CHUNKS):
        scatter(d)
    for d in range(N_CHUNKS - NBUF, N_CHUNKS):
        shandles[d].wait()


def kernel(input_features, table):
    flat_in = input_features.reshape(NW, N_CHUNKS, CHUNK)
    out = _sc_gather(flat_in, table)
    return out.reshape(BATCH, N_FIELDS * OUTPUT_DIM)


# speedup vs baseline: 18.2523x; 3.0223x over previous
import functools

import jax
import jax.numpy as jnp
from jax import lax
from jax.experimental import pallas as pl
from jax.experimental.pallas import tpu as pltpu
from jax.experimental.pallas import tpu_sc as plsc

N_FIELDS = 26
N_PAIRS = N_FIELDS // 2
VALS_PER_FIELD = 32
OUTPUT_DIM = 64
BATCH = 16384

_info = plsc.get_sparse_core_info()
NC, NS, L = _info.num_cores, _info.num_subcores, _info.num_lanes
NW = NC * NS
BB = 128
N_BCHUNK = BATCH // BB
TOT_CHUNKS = N_PAIRS * N_BCHUNK
PER_W = TOT_CHUNKS // NW
NBUF = 4
SKEW = 2


@functools.partial(
    pl.kernel,
    mesh=plsc.VectorSubcoreMesh(core_axis_name="c", subcore_axis_name="s"),
    out_type=jax.ShapeDtypeStruct((BATCH, N_FIELDS * OUTPUT_DIM), jnp.float32),
    scratch_types=[
        pltpu.VMEM((NBUF * 2 * BB,), jnp.int32),
        pltpu.VMEM((NBUF, BB), jnp.int32),
        pltpu.VMEM((NBUF, BB, 2 * OUTPUT_DIM), jnp.float32),
    ]
    + [pltpu.SemaphoreType.DMA] * (3 * NBUF),
    compiler_params=pltpu.CompilerParams(use_tc_tiling_on_sc=True),
)
def _sc_gather(inT_hbm, ptab_hbm, out_hbm, vbuf, idxbuf, rows, *sems):
    vsems = sems[:NBUF]
    gsems = sems[NBUF : 2 * NBUF]
    ssems = sems[2 * NBUF :]
    wid = lax.axis_index("s") * NC + lax.axis_index("c")
    cid0 = wid * PER_W
    lane = lax.iota(jnp.int32, L)

    vhandles = [None] * (PER_W + SKEW)
    ghandles = [None] * PER_W
    shandles = [None] * PER_W

    def chunk_coords(c):
        cid = cid0 + c
        t = cid // N_BCHUNK
        b0 = (cid % N_BCHUNK) * BB
        return t, b0

    def stage_v(c):
        s = c % NBUF
        t, b0 = chunk_coords(c)
        h0 = pltpu.async_copy(
            inT_hbm.at[pl.ds(2 * t * BATCH + b0, BB)],
            vbuf.at[pl.ds((s * 2) * BB, BB)],
            vsems[s],
        )
        h1 = pltpu.async_copy(
            inT_hbm.at[pl.ds((2 * t + 1) * BATCH + b0, BB)],
            vbuf.at[pl.ds((s * 2 + 1) * BB, BB)],
            vsems[s],
        )
        vhandles[c] = (h0, h1)

    def start_gather(c):
        s = c % NBUF
        t, _ = chunk_coords(c)
        for h in vhandles[c]:
            h.wait()
        for i in range(BB // L):
            v0 = vbuf[pl.ds((s * 2) * BB + i * L, L)]
            v1 = vbuf[pl.ds((s * 2 + 1) * BB + i * L, L)]
            c0 = jnp.where((v0 >= 0) & (v0 < VALS_PER_FIELD), v0, 0)
            c1 = jnp.where((v1 >= 0) & (v1 < VALS_PER_FIELD), v1, 0)
            idxbuf[s, pl.ds(i * L, L)] = t * 1024 + c0 * VALS_PER_FIELD + c1
        ghandles[c] = pltpu.async_copy(ptab_hbm.at[idxbuf.at[s]], rows.at[s], gsems[s])

    def start_out(c):
        s = c % NBUF
        t, b0 = chunk_coords(c)
        ghandles[c].wait()
        shandles[c] = pltpu.async_copy(
            rows.at[s], out_hbm.at[pl.ds(b0, BB), pl.ds(t * 2 * OUTPUT_DIM, 2 * OUTPUT_DIM)], ssems[s]
        )

    for c in range(PER_W + SKEW):
        if c < PER_W:
            if c >= NBUF:
                shandles[c - NBUF].wait()
            stage_v(c)
        if SKEW <= c < PER_W + SKEW:
            start_gather(c - SKEW)
        if SKEW + 1 <= c:
            d = c - SKEW - 1
            if 0 <= d < PER_W:
                start_out(d)
    start_out(PER_W - 1)
    for d in range(PER_W - NBUF, PER_W):
        shandles[d].wait()


def kernel(input_features, table):
    inT = input_features.T.reshape(N_FIELDS * BATCH)
    tbl3 = table.reshape(N_PAIRS, 2 * VALS_PER_FIELD, OUTPUT_DIM)
    left = jnp.broadcast_to(
        tbl3[:, :VALS_PER_FIELD, None, :],
        (N_PAIRS, VALS_PER_FIELD, VALS_PER_FIELD, OUTPUT_DIM),
    )
    right = jnp.broadcast_to(
        tbl3[:, None, VALS_PER_FIELD:, :],
        (N_PAIRS, VALS_PER_FIELD, VALS_PER_FIELD, OUTPUT_DIM),
    )
    ptab = jnp.concatenate([left, right], axis=3).reshape(
        N_PAIRS * VALS_PER_FIELD * VALS_PER_FIELD, 2 * OUTPUT_DIM
    )
    return _sc_gather(inT, ptab)


# speedup vs baseline: 18.5283x; 1.0151x over previous
import functools

import jax
import jax.numpy as jnp
from jax import lax
from jax.experimental import pallas as pl
from jax.experimental.pallas import tpu as pltpu
from jax.experimental.pallas import tpu_sc as plsc

N_FIELDS = 26
N_PAIRS = N_FIELDS // 2
VALS_PER_FIELD = 32
OUTPUT_DIM = 64
BATCH = 16384

_info = plsc.get_sparse_core_info()
NC, NS, L = _info.num_cores, _info.num_subcores, _info.num_lanes
NW = NC * NS
BB = 128
N_BCHUNK = BATCH // BB
TOT_CHUNKS = N_PAIRS * N_BCHUNK
PER_W = TOT_CHUNKS // NW
NBUF = 6
SKEW = 3


@functools.partial(
    pl.kernel,
    mesh=plsc.VectorSubcoreMesh(core_axis_name="c", subcore_axis_name="s"),
    out_type=jax.ShapeDtypeStruct((BATCH, N_FIELDS * OUTPUT_DIM), jnp.float32),
    scratch_types=[
        pltpu.VMEM((NBUF * 2 * BB,), jnp.int32),
        pltpu.VMEM((NBUF, BB), jnp.int32),
        pltpu.VMEM((NBUF, BB, 2 * OUTPUT_DIM), jnp.float32),
    ]
    + [pltpu.SemaphoreType.DMA] * (3 * NBUF),
    compiler_params=pltpu.CompilerParams(use_tc_tiling_on_sc=True),
)
def _sc_gather(inT_hbm, ptab_hbm, out_hbm, vbuf, idxbuf, rows, *sems):
    vsems = sems[:NBUF]
    gsems = sems[NBUF : 2 * NBUF]
    ssems = sems[2 * NBUF :]
    wid = lax.axis_index("s") * NC + lax.axis_index("c")
    cid0 = wid * PER_W
    lane = lax.iota(jnp.int32, L)

    vhandles = [None] * (PER_W + SKEW)
    ghandles = [None] * PER_W
    shandles = [None] * PER_W

    def chunk_coords(c):
        cid = cid0 + c
        t = cid // N_BCHUNK
        b0 = (cid % N_BCHUNK) * BB
        return t, b0

    def stage_v(c):
        s = c % NBUF
        t, b0 = chunk_coords(c)
        h0 = pltpu.async_copy(
            inT_hbm.at[pl.ds(2 * t * BATCH + b0, BB)],
            vbuf.at[pl.ds((s * 2) * BB, BB)],
            vsems[s],
        )
        h1 = pltpu.async_copy(
            inT_hbm.at[pl.ds((2 * t + 1) * BATCH + b0, BB)],
            vbuf.at[pl.ds((s * 2 + 1) * BB, BB)],
            vsems[s],
        )
        vhandles[c] = (h0, h1)

    def start_gather(c):
        s = c % NBUF
        t, _ = chunk_coords(c)
        for h in vhandles[c]:
            h.wait()
        for i in range(BB // L):
            v0 = vbuf[pl.ds((s * 2) * BB + i * L, L)]
            v1 = vbuf[pl.ds((s * 2 + 1) * BB + i * L, L)]
            c0 = jnp.where((v0 >= 0) & (v0 < VALS_PER_FIELD), v0, 0)
            c1 = jnp.where((v1 >= 0) & (v1 < VALS_PER_FIELD), v1, 0)
            idxbuf[s, pl.ds(i * L, L)] = t * 1024 + c0 * VALS_PER_FIELD + c1
        ghandles[c] = pltpu.async_copy(ptab_hbm.at[idxbuf.at[s]], rows.at[s], gsems[s])

    def start_out(c):
        s = c % NBUF
        t, b0 = chunk_coords(c)
        ghandles[c].wait()
        shandles[c] = pltpu.async_copy(
            rows.at[s], out_hbm.at[pl.ds(b0, BB), pl.ds(t * 2 * OUTPUT_DIM, 2 * OUTPUT_DIM)], ssems[s]
        )

    for c in range(PER_W + SKEW):
        if c < PER_W:
            if c >= NBUF:
                shandles[c - NBUF].wait()
            stage_v(c)
        if SKEW <= c < PER_W + SKEW:
            start_gather(c - SKEW)
        if SKEW + 1 <= c:
            d = c - SKEW - 1
            if 0 <= d < PER_W:
                start_out(d)
    start_out(PER_W - 1)
    for d in range(PER_W - NBUF, PER_W):
        shandles[d].wait()


def kernel(input_features, table):
    inT = input_features.T.reshape(N_FIELDS * BATCH)
    tbl3 = table.reshape(N_PAIRS, 2 * VALS_PER_FIELD, OUTPUT_DIM)
    left = jnp.broadcast_to(
        tbl3[:, :VALS_PER_FIELD, None, :],
        (N_PAIRS, VALS_PER_FIELD, VALS_PER_FIELD, OUTPUT_DIM),
    )
    right = jnp.broadcast_to(
        tbl3[:, None, VALS_PER_FIELD:, :],
        (N_PAIRS, VALS_PER_FIELD, VALS_PER_FIELD, OUTPUT_DIM),
    )
    ptab = jnp.concatenate([left, right], axis=3).reshape(
        N_PAIRS * VALS_PER_FIELD * VALS_PER_FIELD, 2 * OUTPUT_DIM
    )
    return _sc_gather(inT, ptab)


# speedup vs baseline: 18.6570x; 1.0069x over previous
import functools

import jax
import jax.numpy as jnp
from jax import lax
from jax.experimental import pallas as pl
from jax.experimental.pallas import tpu as pltpu
from jax.experimental.pallas import tpu_sc as plsc

N_FIELDS = 26
N_PAIRS = N_FIELDS // 2
VALS_PER_FIELD = 32
OUTPUT_DIM = 64
BATCH = 16384

_info = plsc.get_sparse_core_info()
NC, NS, L = _info.num_cores, _info.num_subcores, _info.num_lanes
NW = NC * NS
BB = 256
GI = 128
N_BCHUNK = BATCH // BB
TOT_CHUNKS = N_PAIRS * N_BCHUNK
PER_W = TOT_CHUNKS // NW
NBUF = 3
SKEW = 1


@functools.partial(
    pl.kernel,
    mesh=plsc.VectorSubcoreMesh(core_axis_name="c", subcore_axis_name="s"),
    out_type=jax.ShapeDtypeStruct((BATCH, N_FIELDS * OUTPUT_DIM), jnp.float32),
    scratch_types=[
        pltpu.VMEM((NBUF * 2 * BB,), jnp.int32),
        pltpu.VMEM((NBUF * BB,), jnp.int32),
        pltpu.VMEM((NBUF, BB, 2 * OUTPUT_DIM), jnp.float32),
    ]
    + [pltpu.SemaphoreType.DMA] * (3 * NBUF),
    compiler_params=pltpu.CompilerParams(use_tc_tiling_on_sc=True),
)
def _sc_gather(inT_hbm, ptab_hbm, out_hbm, vbuf, idxbuf, rows, *sems):
    vsems = sems[:NBUF]
    gsems = sems[NBUF : 2 * NBUF]
    ssems = sems[2 * NBUF :]
    wid = lax.axis_index("s") * NC + lax.axis_index("c")
    cid0 = wid * PER_W
    lane = lax.iota(jnp.int32, L)

    vhandles = [None] * PER_W
    ghandles = [None] * PER_W
    shandles = [None] * PER_W

    def chunk_coords(c):
        cid = cid0 + c
        t = cid // N_BCHUNK
        b0 = (cid % N_BCHUNK) * BB
        return t, b0

    def stage_v(c):
        s = c % NBUF
        t, b0 = chunk_coords(c)
        h0 = pltpu.async_copy(
            inT_hbm.at[pl.ds(2 * t * BATCH + b0, BB)],
            vbuf.at[pl.ds((s * 2) * BB, BB)],
            vsems[s],
        )
        h1 = pltpu.async_copy(
            inT_hbm.at[pl.ds((2 * t + 1) * BATCH + b0, BB)],
            vbuf.at[pl.ds((s * 2 + 1) * BB, BB)],
            vsems[s],
        )
        vhandles[c] = (h0, h1)

    def start_gather(c):
        s = c % NBUF
        t, _ = chunk_coords(c)
        for h in vhandles[c]:
            h.wait()
        for i in range(BB // L):
            v0 = vbuf[pl.ds((s * 2) * BB + i * L, L)]
            v1 = vbuf[pl.ds((s * 2 + 1) * BB + i * L, L)]
            c0 = jnp.where((v0 >= 0) & (v0 < VALS_PER_FIELD), v0, 0)
            c1 = jnp.where((v1 >= 0) & (v1 < VALS_PER_FIELD), v1, 0)
            idxbuf[pl.ds(s * BB + i * L, L)] = t * 1024 + c0 * VALS_PER_FIELD + c1
        h0 = pltpu.async_copy(
            ptab_hbm.at[idxbuf.at[pl.ds(s * BB, GI)]],
            rows.at[s, pl.ds(0, GI)],
            gsems[s],
        )
        h1 = pltpu.async_copy(
            ptab_hbm.at[idxbuf.at[pl.ds(s * BB + GI, GI)]],
            rows.at[s, pl.ds(GI, GI)],
            gsems[s],
        )
        ghandles[c] = (h0, h1)

    def start_out(c):
        s = c % NBUF
        t, b0 = chunk_coords(c)
        for h in ghandles[c]:
            h.wait()
        shandles[c] = pltpu.async_copy(
            rows.at[s],
            out_hbm.at[pl.ds(b0, BB), pl.ds(t * 2 * OUTPUT_DIM, 2 * OUTPUT_DIM)],
            ssems[s],
        )

    for c in range(PER_W + SKEW):
        if c < PER_W:
            if c >= NBUF:
                shandles[c - NBUF].wait()
            stage_v(c)
        g = c - SKEW
        if 0 <= g < PER_W:
            start_gather(g)
        d = c - SKEW - 1
        if 0 <= d < PER_W:
            start_out(d)
    start_out(PER_W - 1)
    for d in range(PER_W - NBUF, PER_W):
        shandles[d].wait()


def kernel(input_features, table):
    inT = input_features.T.reshape(N_FIELDS * BATCH)
    tbl3 = table.reshape(N_PAIRS, 2 * VALS_PER_FIELD, OUTPUT_DIM)
    left = jnp.broadcast_to(
        tbl3[:, :VALS_PER_FIELD, None, :],
        (N_PAIRS, VALS_PER_FIELD, VALS_PER_FIELD, OUTPUT_DIM),
    )
    right = jnp.broadcast_to(
        tbl3[:, None, VALS_PER_FIELD:, :],
        (N_PAIRS, VALS_PER_FIELD, VALS_PER_FIELD, OUTPUT_DIM),
    )
    ptab = jnp.concatenate([left, right], axis=3).reshape(
        N_PAIRS * VALS_PER_FIELD * VALS_PER_FIELD, 2 * OUTPUT_DIM
    )
    return _sc_gather(inT, ptab)
